# baseline (speedup 1.0000x reference)
import functools

import jax
import jax.numpy as jnp
from jax import lax
from jax.experimental import pallas as pl
from jax.experimental.pallas import tpu as pltpu
from jax.experimental.pallas import tpu_sc as plsc

_LANES = 16
_NUM_WORKERS = 32


def _logits_kernel(x_ref, w_ref, lg_ref):
    lg_ref[...] = jax.lax.dot_general(
        w_ref[...], x_ref[...],
        dimension_numbers=(((1,), (1,)), ((), ())),
        preferred_element_type=jnp.float32,
    )


def _sc_top2_body(lg_hbm, ow_hbm, oi_hbm, buf, owb, oib, *, num_layers,
                  num_experts, tok_per_worker):
    c = lax.axis_index("c")
    s = lax.axis_index("s")
    wid = s * 2 + c
    base = wid * tok_per_worker
    n_chunks = tok_per_worker // _LANES
    neg_inf = jnp.float32(-jnp.inf)
    for l in range(num_layers):
        pltpu.sync_copy(
            lg_hbm.at[pl.ds(l * num_experts, num_experts),
                      pl.ds(base, tok_per_worker)],
            buf)

        def tbody(t, carry):
            sl = pl.ds(t * _LANES, _LANES)
            m1 = buf[0, sl]
            i1 = jnp.zeros((_LANES,), jnp.int32)
            m2 = jnp.full((_LANES,), neg_inf, jnp.float32)
            i2 = jnp.zeros((_LANES,), jnp.int32)
            for e in range(1, num_experts):
                v = buf[e, sl]
                es = jnp.full((_LANES,), e, jnp.int32)
                gt1 = v > m1
                gt2 = v > m2
                m2 = jnp.where(gt1, m1, jnp.where(gt2, v, m2))
                i2 = jnp.where(gt1, i1, jnp.where(gt2, es, i2))
                m1 = jnp.where(gt1, v, m1)
                i1 = jnp.where(gt1, es, i1)
            r = jnp.exp(m2 - m1)
            w1 = 1.0 / (1.0 + r)
            owb[0, sl] = w1
            owb[1, sl] = 1.0 - w1
            oib[0, sl] = i1
            oib[1, sl] = i2
            return carry

        lax.fori_loop(0, n_chunks, tbody, 0)
        pltpu.sync_copy(owb, ow_hbm.at[l].at[:, pl.ds(base, tok_per_worker)])
        pltpu.sync_copy(oib, oi_hbm.at[l].at[:, pl.ds(base, tok_per_worker)])


@jax.jit
def kernel(hidden_states, router_weights):
    t, h = hidden_states.shape
    num_layers, num_experts, _ = router_weights.shape
    rows = num_layers * num_experts
    w2d = router_weights.reshape(rows, h)
    tile = min(1024, t)
    logits = pl.pallas_call(
        _logits_kernel,
        grid=(t // tile,),
        in_specs=[
            pl.BlockSpec((tile, h), lambda i: (i, 0)),
            pl.BlockSpec((rows, h), lambda i: (0, 0)),
        ],
        out_specs=pl.BlockSpec((rows, tile), lambda i: (0, i)),
        out_shape=jax.ShapeDtypeStruct((rows, t), jnp.float32),
    )(hidden_states, w2d)

    tok_per_worker = t // _NUM_WORKERS
    mesh = plsc.VectorSubcoreMesh(core_axis_name="c", subcore_axis_name="s",
                                  num_cores=2, num_subcores=16)
    body = functools.partial(_sc_top2_body, num_layers=num_layers,
                             num_experts=num_experts,
                             tok_per_worker=tok_per_worker)
    ow, oi = pl.kernel(
        body,
        out_type=(
            jax.ShapeDtypeStruct((num_layers, 2, t), jnp.float32),
            jax.ShapeDtypeStruct((num_layers, 2, t), jnp.int32),
        ),
        mesh=mesh,
        scratch_types=[
            pltpu.VMEM((num_experts, tok_per_worker), jnp.float32),
            pltpu.VMEM((2, tok_per_worker), jnp.float32),
            pltpu.VMEM((2, tok_per_worker), jnp.int32),
        ],
    )(logits)
    return jnp.swapaxes(ow, 1, 2), jnp.swapaxes(oi, 1, 2)


# speedup vs baseline: 1.7455x; 1.7455x over previous
import functools

import jax
import jax.numpy as jnp
from jax.experimental import pallas as pl


def _router_kernel(x_ref, w_ref, ow_ref, oi_ref, *, num_layers, num_experts):
    x = x_ref[...]
    w = w_ref[...]
    logits = jax.lax.dot_general(
        w, x,
        dimension_numbers=(((1,), (1,)), ((), ())),
        preferred_element_type=jnp.float32,
    )
    tile = x.shape[0]
    iota = jax.lax.broadcasted_iota(jnp.int32, (num_experts, tile), 0)
    neg_inf = jnp.float32(-jnp.inf)
    for l in range(num_layers):
        lg = logits[l * num_experts:(l + 1) * num_experts, :]
        l1 = jnp.max(lg, axis=0)
        i1 = jnp.min(jnp.where(lg == l1[None, :], iota, num_experts), axis=0)
        masked = jnp.where(iota == i1[None, :], neg_inf, lg)
        l2 = jnp.max(masked, axis=0)
        i2 = jnp.min(jnp.where(masked == l2[None, :], iota, num_experts),
                     axis=0)
        r = jnp.exp(l2 - l1)
        w1 = 1.0 / (1.0 + r)
        ow_ref[l, 0, :] = w1
        ow_ref[l, 1, :] = 1.0 - w1
        oi_ref[l, 0, :] = i1.astype(jnp.int32)
        oi_ref[l, 1, :] = i2.astype(jnp.int32)


@jax.jit
def kernel(hidden_states, router_weights):
    t, h = hidden_states.shape
    num_layers, num_experts, _ = router_weights.shape
    w2d = router_weights.reshape(num_layers * num_experts, h)
    tile = min(1024, t)
    grid = (t // tile,)
    kfn = functools.partial(_router_kernel, num_layers=num_layers,
                            num_experts=num_experts)
    ow, oi = pl.pallas_call(
        kfn,
        grid=grid,
        in_specs=[
            pl.BlockSpec((tile, h), lambda i: (i, 0)),
            pl.BlockSpec((num_layers * num_experts, h), lambda i: (0, 0)),
        ],
        out_specs=[
            pl.BlockSpec((num_layers, 2, tile), lambda i: (0, 0, i)),
            pl.BlockSpec((num_layers, 2, tile), lambda i: (0, 0, i)),
        ],
        out_shape=[
            jax.ShapeDtypeStruct((num_layers, 2, t), jnp.float32),
            jax.ShapeDtypeStruct((num_layers, 2, t), jnp.int32),
        ],
    )(hidden_states, w2d)
    return jnp.swapaxes(ow, 1, 2), jnp.swapaxes(oi, 1, 2)
